# baseline (speedup 1.0000x reference)
import functools

import jax
import jax.numpy as jnp
from jax import lax
from jax.experimental import pallas as pl
from jax.experimental.pallas import tpu as pltpu
from jax.experimental.pallas import tpu_sc as plsc

_N = 10000
_E = 320000
_D = 128
_G = 64
_NC = 2
_NS = 16
_NW = _NC * _NS
_EPW = _E // _NW
_K = 128
_NCH = 80
_EPP = _NCH * _K
_NCHH = _NCH // 2
_NP = 10112
_RPT = _NP // _NS
_BLK = 1000
_NBLK = _N // _BLK


def _sc_mesh():
    return plsc.VectorSubcoreMesh(
        core_axis_name="c", subcore_axis_name="s",
        num_cores=_NC, num_subcores=_NS)


def _sc_degree(dst_r, ones_rows, zeros128):
    @functools.partial(
        pl.kernel,
        out_type=jax.ShapeDtypeStruct((_NC, _NP, _D), jnp.float32),
        mesh=_sc_mesh(),
        scratch_types=[
            pltpu.VMEM((_NCH, _K), jnp.int32),
            pltpu.VMEM((_K, _D), jnp.float32),
            pltpu.VMEM_SHARED((_NP, _D), jnp.float32),
        ],
    )
    def deg_kernel(dst_hbm, ones_hbm, zeros_hbm, out_hbm, dst_v, ones_v, deg_sh):
        c = lax.axis_index("c")
        s = lax.axis_index("s")
        w = s * _NC + c
        pltpu.sync_copy(dst_hbm.at[w], dst_v)
        pltpu.sync_copy(ones_hbm, ones_v)
        pltpu.sync_copy(zeros_hbm, deg_sh.at[pl.ds(s * _RPT, _RPT)])
        plsc.subcore_barrier()

        def body(j, carry):
            pltpu.sync_copy(ones_v, deg_sh.at[dst_v.at[j]], add=True)
            return carry

        lax.fori_loop(0, _NCH, body, 0)
        plsc.subcore_barrier()
        pltpu.sync_copy(deg_sh.at[pl.ds(s * _RPT, _RPT)],
                        out_hbm.at[c, pl.ds(s * _RPT, _RPT)])

    return deg_kernel(dst_r, ones_rows, zeros128)


def _sc_scatter(y, src_r, dst_r, zeros128):
    @functools.partial(
        pl.kernel,
        out_type=jax.ShapeDtypeStruct((_NC, _NP, _D), jnp.float32),
        mesh=_sc_mesh(),
        scratch_types=[
            pltpu.VMEM((_NCHH, _K), jnp.int32),
            pltpu.VMEM((_NCHH, _K), jnp.int32),
            pltpu.VMEM((_K, _D), jnp.float32),
            pltpu.VMEM((_K, _D), jnp.float32),
            pltpu.SemaphoreType.DMA,
            pltpu.SemaphoreType.DMA,
            pltpu.VMEM_SHARED((_NP, _D), jnp.float32),
        ],
    )
    def scat_kernel(y_hbm, src_hbm, dst_hbm, zeros_hbm, out_hbm,
                    src_v, dst_v, rows0, rows1, sem0, sem1, z_sh):
        c = lax.axis_index("c")
        s = lax.axis_index("s")
        w = s * _NC + c
        pltpu.sync_copy(zeros_hbm, z_sh.at[pl.ds(s * _RPT, _RPT)])
        plsc.subcore_barrier()

        for h in range(2):
            pltpu.sync_copy(src_hbm.at[w, pl.ds(h * _NCHH, _NCHH)], src_v)
            pltpu.sync_copy(dst_hbm.at[w, pl.ds(h * _NCHH, _NCHH)], dst_v)

            def body(j, carry):
                pltpu.async_copy(y_hbm.at[src_v.at[j]], rows0, sem0).wait()
                pltpu.sync_copy(rows0, z_sh.at[dst_v.at[j]], add=True)
                return carry

            lax.fori_loop(0, _NCHH, body, 0)
        plsc.subcore_barrier()
        pltpu.sync_copy(z_sh.at[pl.ds(s * _RPT, _RPT)],
                        out_hbm.at[c, pl.ds(s * _RPT, _RPT)])

    return scat_kernel(y, src_r, dst_r, zeros128)


def _dinv_block(deg_ref):
    return lax.rsqrt(deg_ref[0, :, 0:1] + deg_ref[1, :, 0:1] + 1.0)


def _tc_dense1(x, deg, W1, b1, Wg1):
    def body(x_ref, deg_ref, W1_ref, b1_ref, Wg1_ref, y_ref):
        dinv = _dinv_block(deg_ref)
        h = jnp.maximum(
            jnp.dot(x_ref[...], W1_ref[...],
                    preferred_element_type=jnp.float32) + b1_ref[...], 0.0)
        y_ref[...] = dinv * jnp.dot(h, Wg1_ref[...],
                                    preferred_element_type=jnp.float32)

    return pl.pallas_call(
        body,
        grid=(_NBLK,),
        in_specs=[
            pl.BlockSpec((_BLK, _D), lambda i: (i, 0)),
            pl.BlockSpec((_NC, _BLK, _D), lambda i: (0, i, 0)),
            pl.BlockSpec((_D, _D), lambda i: (0, 0)),
            pl.BlockSpec((1, _D), lambda i: (0, 0)),
            pl.BlockSpec((_D, _D), lambda i: (0, 0)),
        ],
        out_specs=pl.BlockSpec((_BLK, _D), lambda i: (i, 0)),
        out_shape=jax.ShapeDtypeStruct((_N, _D), jnp.float32),
    )(x, deg, W1, b1, Wg1)


def _tc_mid(p, yprev, deg, bg, Wgn):
    def body(p_ref, y_ref, deg_ref, bg_ref, Wg_ref, o_ref):
        dinv = _dinv_block(deg_ref)
        z = p_ref[0] + p_ref[1] + y_ref[...]
        h = jnp.maximum(dinv * z + bg_ref[...], 0.0)
        o_ref[...] = dinv * jnp.dot(h, Wg_ref[...],
                                    preferred_element_type=jnp.float32)

    return pl.pallas_call(
        body,
        grid=(_NBLK,),
        in_specs=[
            pl.BlockSpec((_NC, _BLK, _D), lambda i: (0, i, 0)),
            pl.BlockSpec((_BLK, _D), lambda i: (i, 0)),
            pl.BlockSpec((_NC, _BLK, _D), lambda i: (0, i, 0)),
            pl.BlockSpec((1, _D), lambda i: (0, 0)),
            pl.BlockSpec((_D, _D), lambda i: (0, 0)),
        ],
        out_specs=pl.BlockSpec((_BLK, _D), lambda i: (i, 0)),
        out_shape=jax.ShapeDtypeStruct((_N, _D), jnp.float32),
    )(p, yprev, deg, bg, Wgn)


def _tc_pool(p, y3, deg, bg3, batch_r, W2, b2):
    def body(p_ref, y_ref, deg_ref, bg_ref, b_ref, W2_ref, b2_ref,
             o_ref, sums, cnts):
        i = pl.program_id(0)

        @pl.when(i == 0)
        def _():
            sums[...] = jnp.zeros_like(sums)
            cnts[...] = jnp.zeros_like(cnts)

        dinv = _dinv_block(deg_ref)
        h = jnp.maximum(dinv * (p_ref[0] + p_ref[1] + y_ref[...])
                        + bg_ref[...], 0.0)
        gids = lax.broadcasted_iota(jnp.int32, (_G, _BLK), 0)
        onehot = (gids == b_ref[0]).astype(jnp.float32)
        sums[...] += jnp.dot(onehot, h, preferred_element_type=jnp.float32)
        cnts[...] += jnp.broadcast_to(
            jnp.sum(onehot, axis=1, keepdims=True), (_G, _D))

        @pl.when(i == _NBLK - 1)
        def _():
            pooled = sums[...] / jnp.maximum(cnts[...], 1.0)
            o_ref[...] = jnp.dot(pooled, W2_ref[...],
                                 preferred_element_type=jnp.float32) + b2_ref[...]

    return pl.pallas_call(
        body,
        grid=(_NBLK,),
        in_specs=[
            pl.BlockSpec((_NC, _BLK, _D), lambda i: (0, i, 0)),
            pl.BlockSpec((_BLK, _D), lambda i: (i, 0)),
            pl.BlockSpec((_NC, _BLK, _D), lambda i: (0, i, 0)),
            pl.BlockSpec((1, _D), lambda i: (0, 0)),
            pl.BlockSpec((1, 1, _BLK), lambda i: (i, 0, 0)),
            pl.BlockSpec((_D, _D), lambda i: (0, 0)),
            pl.BlockSpec((1, _D), lambda i: (0, 0)),
        ],
        out_specs=pl.BlockSpec((_G, _D), lambda i: (0, 0)),
        out_shape=jax.ShapeDtypeStruct((_G, _D), jnp.float32),
        scratch_shapes=[
            pltpu.VMEM((_G, _D), jnp.float32),
            pltpu.VMEM((_G, _D), jnp.float32),
        ],
    )(p, y3, deg, bg3, batch_r, W2, b2)


def kernel(x, edge_index, batch, W1, b1, Wg1, bg1, Wg2, bg2, Wg3, bg3, W2, b2):
    src = edge_index[0].reshape(_NW, _EPW)
    dst = edge_index[1].reshape(_NW, _EPW)
    npad = _EPP - _EPW
    src = jnp.concatenate(
        [src, jnp.zeros((_NW, npad), jnp.int32)], axis=1).reshape(_NW, _NCH, _K)
    dst = jnp.concatenate(
        [dst, jnp.full((_NW, npad), _N, jnp.int32)], axis=1).reshape(_NW, _NCH, _K)
    zeros128 = jnp.zeros((_RPT, _D), jnp.float32)
    ones128 = jnp.ones((_K, _D), jnp.float32)
    batch_r = batch.reshape(_NBLK, 1, _BLK)
    b1r = b1.reshape(1, _D)
    bg1r = bg1.reshape(1, _D)
    bg2r = bg2.reshape(1, _D)
    bg3r = bg3.reshape(1, _D)
    b2r = b2.reshape(1, _D)

    deg = _sc_degree(dst, ones128, zeros128)
    y1 = _tc_dense1(x, deg, W1, b1r, Wg1)
    p1 = _sc_scatter(y1, src, dst, zeros128)
    y2 = _tc_mid(p1, y1, deg, bg1r, Wg2)
    p2 = _sc_scatter(y2, src, dst, zeros128)
    y3 = _tc_mid(p2, y2, deg, bg2r, Wg3)
    p3 = _sc_scatter(y3, src, dst, zeros128)
    return _tc_pool(p3, y3, deg, bg3r, batch_r, W2, b2r)


# speedup vs baseline: 2.2235x; 2.2235x over previous
import functools

import jax
import jax.numpy as jnp
from jax import lax
from jax.experimental import pallas as pl
from jax.experimental.pallas import tpu as pltpu
from jax.experimental.pallas import tpu_sc as plsc

_N = 10000
_E = 320000
_D = 128
_G = 64
_NC = 2
_NS = 16
_NW = _NC * _NS
_EPW = _E // _NW
_K = 125
_NCH = _EPW // _K
_NCHH = _NCH // 2
_NP = 10112
_RPT = _NP // _NS
_BLK = 1000
_NBLK = _N // _BLK


def _sc_mesh():
    return plsc.VectorSubcoreMesh(
        core_axis_name="c", subcore_axis_name="s",
        num_cores=_NC, num_subcores=_NS)


def _sc_degree(dst_r, ones_rows, zeros128):
    @functools.partial(
        pl.kernel,
        out_type=jax.ShapeDtypeStruct((_NC, _NP, _D), jnp.float32),
        mesh=_sc_mesh(),
        scratch_types=[
            pltpu.VMEM((_NCH, _K), jnp.int32),
            pltpu.VMEM((_K, _D), jnp.float32),
            pltpu.VMEM_SHARED((_NP, _D), jnp.float32),
        ],
    )
    def deg_kernel(dst_hbm, ones_hbm, zeros_hbm, out_hbm, dst_v, ones_v, deg_sh):
        c = lax.axis_index("c")
        s = lax.axis_index("s")
        w = s * _NC + c
        pltpu.sync_copy(dst_hbm.at[w], dst_v)
        pltpu.sync_copy(ones_hbm, ones_v)
        pltpu.sync_copy(zeros_hbm, deg_sh.at[pl.ds(s * _RPT, _RPT)])
        plsc.subcore_barrier()

        def body(j, carry):
            pltpu.sync_copy(ones_v, deg_sh.at[dst_v.at[j]], add=True)
            return carry

        lax.fori_loop(0, _NCH, body, 0)
        plsc.subcore_barrier()
        pltpu.sync_copy(deg_sh.at[pl.ds(s * _RPT, _RPT)],
                        out_hbm.at[c, pl.ds(s * _RPT, _RPT)])

    return deg_kernel(dst_r, ones_rows, zeros128)


def _sc_scatter(y, src_r, dst_r, zeros128):
    @functools.partial(
        pl.kernel,
        out_type=jax.ShapeDtypeStruct((_NC, _NP, _D), jnp.float32),
        mesh=_sc_mesh(),
        scratch_types=[
            pltpu.VMEM((_NCHH, _K), jnp.int32),
            pltpu.VMEM((_NCHH, _K), jnp.int32),
            pltpu.VMEM((_K, _D), jnp.float32),
            pltpu.VMEM((_K, _D), jnp.float32),
            pltpu.SemaphoreType.DMA,
            pltpu.SemaphoreType.DMA,
            pltpu.VMEM_SHARED((_NP, _D), jnp.float32),
        ],
    )
    def scat_kernel(y_hbm, src_hbm, dst_hbm, zeros_hbm, out_hbm,
                    src_v, dst_v, rows0, rows1, sem0, sem1, z_sh):
        c = lax.axis_index("c")
        s = lax.axis_index("s")
        w = s * _NC + c
        pltpu.sync_copy(zeros_hbm, z_sh.at[pl.ds(s * _RPT, _RPT)])
        plsc.subcore_barrier()

        for h in range(2):
            pltpu.sync_copy(src_hbm.at[w, pl.ds(h * _NCHH, _NCHH)], src_v)
            pltpu.sync_copy(dst_hbm.at[w, pl.ds(h * _NCHH, _NCHH)], dst_v)

            def body(j, carry):
                pltpu.async_copy(y_hbm.at[src_v.at[j]], rows0, sem0).wait()
                pltpu.sync_copy(rows0, z_sh.at[dst_v.at[j]], add=True)
                return carry

            lax.fori_loop(0, _NCHH, body, 0)
        plsc.subcore_barrier()
        pltpu.sync_copy(z_sh.at[pl.ds(s * _RPT, _RPT)],
                        out_hbm.at[c, pl.ds(s * _RPT, _RPT)])

    return scat_kernel(y, src_r, dst_r, zeros128)


def _dinv_block(deg_ref):
    return lax.rsqrt(deg_ref[0, :, 0:1] + deg_ref[1, :, 0:1] + 1.0)


def _tc_dense1(x, deg, W1, b1, Wg1):
    def body(x_ref, deg_ref, W1_ref, b1_ref, Wg1_ref, y_ref):
        dinv = _dinv_block(deg_ref)
        h = jnp.maximum(
            jnp.dot(x_ref[...], W1_ref[...],
                    preferred_element_type=jnp.float32) + b1_ref[...], 0.0)
        y_ref[...] = dinv * jnp.dot(h, Wg1_ref[...],
                                    preferred_element_type=jnp.float32)

    return pl.pallas_call(
        body,
        grid=(_NBLK,),
        in_specs=[
            pl.BlockSpec((_BLK, _D), lambda i: (i, 0)),
            pl.BlockSpec((_NC, _BLK, _D), lambda i: (0, i, 0)),
            pl.BlockSpec((_D, _D), lambda i: (0, 0)),
            pl.BlockSpec((1, _D), lambda i: (0, 0)),
            pl.BlockSpec((_D, _D), lambda i: (0, 0)),
        ],
        out_specs=pl.BlockSpec((_BLK, _D), lambda i: (i, 0)),
        out_shape=jax.ShapeDtypeStruct((_N, _D), jnp.float32),
    )(x, deg, W1, b1, Wg1)


def _tc_mid(p, yprev, deg, bg, Wgn):
    def body(p_ref, y_ref, deg_ref, bg_ref, Wg_ref, o_ref):
        dinv = _dinv_block(deg_ref)
        z = p_ref[0] + p_ref[1] + y_ref[...]
        h = jnp.maximum(dinv * z + bg_ref[...], 0.0)
        o_ref[...] = dinv * jnp.dot(h, Wg_ref[...],
                                    preferred_element_type=jnp.float32)

    return pl.pallas_call(
        body,
        grid=(_NBLK,),
        in_specs=[
            pl.BlockSpec((_NC, _BLK, _D), lambda i: (0, i, 0)),
            pl.BlockSpec((_BLK, _D), lambda i: (i, 0)),
            pl.BlockSpec((_NC, _BLK, _D), lambda i: (0, i, 0)),
            pl.BlockSpec((1, _D), lambda i: (0, 0)),
            pl.BlockSpec((_D, _D), lambda i: (0, 0)),
        ],
        out_specs=pl.BlockSpec((_BLK, _D), lambda i: (i, 0)),
        out_shape=jax.ShapeDtypeStruct((_N, _D), jnp.float32),
    )(p, yprev, deg, bg, Wgn)


def _tc_pool(p, y3, deg, bg3, batch_r, W2, b2):
    def body(p_ref, y_ref, deg_ref, bg_ref, b_ref, W2_ref, b2_ref,
             o_ref, sums, cnts):
        i = pl.program_id(0)

        @pl.when(i == 0)
        def _():
            sums[...] = jnp.zeros_like(sums)
            cnts[...] = jnp.zeros_like(cnts)

        dinv = _dinv_block(deg_ref)
        h = jnp.maximum(dinv * (p_ref[0] + p_ref[1] + y_ref[...])
                        + bg_ref[...], 0.0)
        gids = lax.broadcasted_iota(jnp.int32, (_G, _BLK), 0)
        onehot = (gids == b_ref[0]).astype(jnp.float32)
        sums[...] += jnp.dot(onehot, h, preferred_element_type=jnp.float32)
        cnts[...] += jnp.broadcast_to(
            jnp.sum(onehot, axis=1, keepdims=True), (_G, _D))

        @pl.when(i == _NBLK - 1)
        def _():
            pooled = sums[...] / jnp.maximum(cnts[...], 1.0)
            o_ref[...] = jnp.dot(pooled, W2_ref[...],
                                 preferred_element_type=jnp.float32) + b2_ref[...]

    return pl.pallas_call(
        body,
        grid=(_NBLK,),
        in_specs=[
            pl.BlockSpec((_NC, _BLK, _D), lambda i: (0, i, 0)),
            pl.BlockSpec((_BLK, _D), lambda i: (i, 0)),
            pl.BlockSpec((_NC, _BLK, _D), lambda i: (0, i, 0)),
            pl.BlockSpec((1, _D), lambda i: (0, 0)),
            pl.BlockSpec((1, 1, _BLK), lambda i: (i, 0, 0)),
            pl.BlockSpec((_D, _D), lambda i: (0, 0)),
            pl.BlockSpec((1, _D), lambda i: (0, 0)),
        ],
        out_specs=pl.BlockSpec((_G, _D), lambda i: (0, 0)),
        out_shape=jax.ShapeDtypeStruct((_G, _D), jnp.float32),
        scratch_shapes=[
            pltpu.VMEM((_G, _D), jnp.float32),
            pltpu.VMEM((_G, _D), jnp.float32),
        ],
    )(p, y3, deg, bg3, batch_r, W2, b2)


def kernel(x, edge_index, batch, W1, b1, Wg1, bg1, Wg2, bg2, Wg3, bg3, W2, b2):
    src = edge_index[0].reshape(_NW, _NCH, _K)
    dst = edge_index[1].reshape(_NW, _NCH, _K)
    zeros128 = jnp.zeros((_RPT, _D), jnp.float32)
    ones128 = jnp.ones((_K, _D), jnp.float32)
    batch_r = batch.reshape(_NBLK, 1, _BLK)
    b1r = b1.reshape(1, _D)
    bg1r = bg1.reshape(1, _D)
    bg2r = bg2.reshape(1, _D)
    bg3r = bg3.reshape(1, _D)
    b2r = b2.reshape(1, _D)

    deg = _sc_degree(dst, ones128, zeros128)
    y1 = _tc_dense1(x, deg, W1, b1r, Wg1)
    p1 = _sc_scatter(y1, src, dst, zeros128)
    y2 = _tc_mid(p1, y1, deg, bg1r, Wg2)
    p2 = _sc_scatter(y2, src, dst, zeros128)
    y3 = _tc_mid(p2, y2, deg, bg2r, Wg3)
    p3 = _sc_scatter(y3, src, dst, zeros128)
    return _tc_pool(p3, y3, deg, bg3r, batch_r, W2, b2r)


# speedup vs baseline: 2.5118x; 1.1297x over previous
import functools

import jax
import jax.numpy as jnp
from jax import lax
from jax.experimental import pallas as pl
from jax.experimental.pallas import tpu as pltpu
from jax.experimental.pallas import tpu_sc as plsc

_N = 10000
_E = 320000
_D = 128
_G = 64
_NC = 2
_NS = 16
_NW = _NC * _NS
_EPW = _E // _NW
_K = 125
_NCH = _EPW // _K
_NCHH = _NCH // 2
_NP = 10112
_RPT = _NP // _NS
_BLK = 1000
_NBLK = _N // _BLK


def _sc_mesh():
    return plsc.VectorSubcoreMesh(
        core_axis_name="c", subcore_axis_name="s",
        num_cores=_NC, num_subcores=_NS)


def _sc_degree(dst_r, ones_rows, zeros128):
    @functools.partial(
        pl.kernel,
        out_type=jax.ShapeDtypeStruct((_NC, _NP, _D), jnp.float32),
        mesh=_sc_mesh(),
        scratch_types=[
            pltpu.VMEM((_NCH, _K), jnp.int32),
            pltpu.VMEM((_K, _D), jnp.float32),
            pltpu.VMEM_SHARED((_NP, _D), jnp.float32),
        ],
    )
    def deg_kernel(dst_hbm, ones_hbm, zeros_hbm, out_hbm, dst_v, ones_v, deg_sh):
        c = lax.axis_index("c")
        s = lax.axis_index("s")
        w = s * _NC + c
        pltpu.sync_copy(dst_hbm.at[w], dst_v)
        pltpu.sync_copy(ones_hbm, ones_v)
        pltpu.sync_copy(zeros_hbm, deg_sh.at[pl.ds(s * _RPT, _RPT)])
        plsc.subcore_barrier()

        def body(j, carry):
            pltpu.sync_copy(ones_v, deg_sh.at[dst_v.at[j]], add=True)
            return carry

        lax.fori_loop(0, _NCH, body, 0)
        plsc.subcore_barrier()
        pltpu.sync_copy(deg_sh.at[pl.ds(s * _RPT, _RPT)],
                        out_hbm.at[c, pl.ds(s * _RPT, _RPT)])

    return deg_kernel(dst_r, ones_rows, zeros128)


def _sc_scatter(y, src_r, dst_r, zeros128):
    @functools.partial(
        pl.kernel,
        out_type=jax.ShapeDtypeStruct((_NC, _NP, _D), jnp.float32),
        mesh=_sc_mesh(),
        scratch_types=[
            pltpu.VMEM((_NCHH, _K), jnp.int32),
            pltpu.VMEM((_NCHH, _K), jnp.int32),
            pltpu.VMEM((_K, _D), jnp.float32),
            pltpu.VMEM((_K, _D), jnp.float32),
            pltpu.SemaphoreType.DMA,
            pltpu.SemaphoreType.DMA,
            pltpu.VMEM_SHARED((_NP, _D), jnp.float32),
        ],
    )
    def scat_kernel(y_hbm, src_hbm, dst_hbm, zeros_hbm, out_hbm,
                    src_v, dst_v, rows0, rows1, sem0, sem1, z_sh):
        c = lax.axis_index("c")
        s = lax.axis_index("s")
        w = s * _NC + c
        pltpu.sync_copy(zeros_hbm, z_sh.at[pl.ds(s * _RPT, _RPT)])
        plsc.subcore_barrier()

        for h in range(2):
            pltpu.sync_copy(src_hbm.at[w, pl.ds(h * _NCHH, _NCHH)], src_v)
            pltpu.sync_copy(dst_hbm.at[w, pl.ds(h * _NCHH, _NCHH)], dst_v)

            def body(jj, carry):
                j0 = 2 * jj
                j1 = j0 + 1
                d0 = pltpu.async_copy(y_hbm.at[src_v.at[j0]], rows0, sem0)
                d1 = pltpu.async_copy(y_hbm.at[src_v.at[j1]], rows1, sem1)
                d0.wait()
                pltpu.sync_copy(rows0, z_sh.at[dst_v.at[j0]], add=True)
                d1.wait()
                pltpu.sync_copy(rows1, z_sh.at[dst_v.at[j1]], add=True)
                return carry

            lax.fori_loop(0, _NCHH // 2, body, 0)
        plsc.subcore_barrier()
        pltpu.sync_copy(z_sh.at[pl.ds(s * _RPT, _RPT)],
                        out_hbm.at[c, pl.ds(s * _RPT, _RPT)])

    return scat_kernel(y, src_r, dst_r, zeros128)


def _dinv_block(deg_ref):
    return lax.rsqrt(deg_ref[0, :, 0:1] + deg_ref[1, :, 0:1] + 1.0)


def _tc_dense1(x, deg, W1, b1, Wg1):
    def body(x_ref, deg_ref, W1_ref, b1_ref, Wg1_ref, y_ref):
        dinv = _dinv_block(deg_ref)
        h = jnp.maximum(
            jnp.dot(x_ref[...], W1_ref[...],
                    preferred_element_type=jnp.float32) + b1_ref[...], 0.0)
        y_ref[...] = dinv * jnp.dot(h, Wg1_ref[...],
                                    preferred_element_type=jnp.float32)

    return pl.pallas_call(
        body,
        grid=(_NBLK,),
        in_specs=[
            pl.BlockSpec((_BLK, _D), lambda i: (i, 0)),
            pl.BlockSpec((_NC, _BLK, _D), lambda i: (0, i, 0)),
            pl.BlockSpec((_D, _D), lambda i: (0, 0)),
            pl.BlockSpec((1, _D), lambda i: (0, 0)),
            pl.BlockSpec((_D, _D), lambda i: (0, 0)),
        ],
        out_specs=pl.BlockSpec((_BLK, _D), lambda i: (i, 0)),
        out_shape=jax.ShapeDtypeStruct((_N, _D), jnp.float32),
    )(x, deg, W1, b1, Wg1)


def _tc_mid(p, yprev, deg, bg, Wgn):
    def body(p_ref, y_ref, deg_ref, bg_ref, Wg_ref, o_ref):
        dinv = _dinv_block(deg_ref)
        z = p_ref[0] + p_ref[1] + y_ref[...]
        h = jnp.maximum(dinv * z + bg_ref[...], 0.0)
        o_ref[...] = dinv * jnp.dot(h, Wg_ref[...],
                                    preferred_element_type=jnp.float32)

    return pl.pallas_call(
        body,
        grid=(_NBLK,),
        in_specs=[
            pl.BlockSpec((_NC, _BLK, _D), lambda i: (0, i, 0)),
            pl.BlockSpec((_BLK, _D), lambda i: (i, 0)),
            pl.BlockSpec((_NC, _BLK, _D), lambda i: (0, i, 0)),
            pl.BlockSpec((1, _D), lambda i: (0, 0)),
            pl.BlockSpec((_D, _D), lambda i: (0, 0)),
        ],
        out_specs=pl.BlockSpec((_BLK, _D), lambda i: (i, 0)),
        out_shape=jax.ShapeDtypeStruct((_N, _D), jnp.float32),
    )(p, yprev, deg, bg, Wgn)


def _tc_pool(p, y3, deg, bg3, batch_r, W2, b2):
    def body(p_ref, y_ref, deg_ref, bg_ref, b_ref, W2_ref, b2_ref,
             o_ref, sums, cnts):
        i = pl.program_id(0)

        @pl.when(i == 0)
        def _():
            sums[...] = jnp.zeros_like(sums)
            cnts[...] = jnp.zeros_like(cnts)

        dinv = _dinv_block(deg_ref)
        h = jnp.maximum(dinv * (p_ref[0] + p_ref[1] + y_ref[...])
                        + bg_ref[...], 0.0)
        gids = lax.broadcasted_iota(jnp.int32, (_G, _BLK), 0)
        onehot = (gids == b_ref[0]).astype(jnp.float32)
        sums[...] += jnp.dot(onehot, h, preferred_element_type=jnp.float32)
        cnts[...] += jnp.broadcast_to(
            jnp.sum(onehot, axis=1, keepdims=True), (_G, _D))

        @pl.when(i == _NBLK - 1)
        def _():
            pooled = sums[...] / jnp.maximum(cnts[...], 1.0)
            o_ref[...] = jnp.dot(pooled, W2_ref[...],
                                 preferred_element_type=jnp.float32) + b2_ref[...]

    return pl.pallas_call(
        body,
        grid=(_NBLK,),
        in_specs=[
            pl.BlockSpec((_NC, _BLK, _D), lambda i: (0, i, 0)),
            pl.BlockSpec((_BLK, _D), lambda i: (i, 0)),
            pl.BlockSpec((_NC, _BLK, _D), lambda i: (0, i, 0)),
            pl.BlockSpec((1, _D), lambda i: (0, 0)),
            pl.BlockSpec((1, 1, _BLK), lambda i: (i, 0, 0)),
            pl.BlockSpec((_D, _D), lambda i: (0, 0)),
            pl.BlockSpec((1, _D), lambda i: (0, 0)),
        ],
        out_specs=pl.BlockSpec((_G, _D), lambda i: (0, 0)),
        out_shape=jax.ShapeDtypeStruct((_G, _D), jnp.float32),
        scratch_shapes=[
            pltpu.VMEM((_G, _D), jnp.float32),
            pltpu.VMEM((_G, _D), jnp.float32),
        ],
    )(p, y3, deg, bg3, batch_r, W2, b2)


def kernel(x, edge_index, batch, W1, b1, Wg1, bg1, Wg2, bg2, Wg3, bg3, W2, b2):
    src = edge_index[0].reshape(_NW, _NCH, _K)
    dst = edge_index[1].reshape(_NW, _NCH, _K)
    zeros128 = jnp.zeros((_RPT, _D), jnp.float32)
    ones128 = jnp.ones((_K, _D), jnp.float32)
    batch_r = batch.reshape(_NBLK, 1, _BLK)
    b1r = b1.reshape(1, _D)
    bg1r = bg1.reshape(1, _D)
    bg2r = bg2.reshape(1, _D)
    bg3r = bg3.reshape(1, _D)
    b2r = b2.reshape(1, _D)

    deg = _sc_degree(dst, ones128, zeros128)
    y1 = _tc_dense1(x, deg, W1, b1r, Wg1)
    p1 = _sc_scatter(y1, src, dst, zeros128)
    y2 = _tc_mid(p1, y1, deg, bg1r, Wg2)
    p2 = _sc_scatter(y2, src, dst, zeros128)
    y3 = _tc_mid(p2, y2, deg, bg2r, Wg3)
    p3 = _sc_scatter(y3, src, dst, zeros128)
    return _tc_pool(p3, y3, deg, bg3r, batch_r, W2, b2r)


# speedup vs baseline: 2.7690x; 1.1024x over previous
import functools

import jax
import jax.numpy as jnp
from jax import lax
from jax.experimental import pallas as pl
from jax.experimental.pallas import tpu as pltpu
from jax.experimental.pallas import tpu_sc as plsc

_N = 10000
_E = 320000
_D = 128
_G = 64
_NC = 2
_NS = 16
_NW = _NC * _NS
_EPW = _E // _NW
_K = 125
_NCH = _EPW // _K
_NCHH = _NCH // 2
_NP = 10112
_RPT = _NP // _NS
_BLK = 1000
_NBLK = _N // _BLK


def _sc_mesh():
    return plsc.VectorSubcoreMesh(
        core_axis_name="c", subcore_axis_name="s",
        num_cores=_NC, num_subcores=_NS)


def _sc_degree(dst_r, ones_rows, zeros128):
    @functools.partial(
        pl.kernel,
        out_type=jax.ShapeDtypeStruct((_NC, _NP, _D), jnp.float32),
        mesh=_sc_mesh(),
        scratch_types=[
            pltpu.VMEM((_NCH, _K), jnp.int32),
            pltpu.VMEM((_K, _D), jnp.float32),
            pltpu.VMEM_SHARED((_NP, _D), jnp.float32),
        ],
    )
    def deg_kernel(dst_hbm, ones_hbm, zeros_hbm, out_hbm, dst_v, ones_v, deg_sh):
        c = lax.axis_index("c")
        s = lax.axis_index("s")
        w = s * _NC + c
        pltpu.sync_copy(dst_hbm.at[w], dst_v)
        pltpu.sync_copy(ones_hbm, ones_v)
        pltpu.sync_copy(zeros_hbm, deg_sh.at[pl.ds(s * _RPT, _RPT)])
        plsc.subcore_barrier()

        def body(j, carry):
            pltpu.sync_copy(ones_v, deg_sh.at[dst_v.at[j]], add=True)
            return carry

        lax.fori_loop(0, _NCH, body, 0)
        plsc.subcore_barrier()
        pltpu.sync_copy(deg_sh.at[pl.ds(s * _RPT, _RPT)],
                        out_hbm.at[c, pl.ds(s * _RPT, _RPT)])

    return deg_kernel(dst_r, ones_rows, zeros128)


def _sc_scatter(y, src_r, dst_r, zeros128):
    @functools.partial(
        pl.kernel,
        out_type=jax.ShapeDtypeStruct((_NC, _NP, _D), jnp.float32),
        mesh=_sc_mesh(),
        scratch_types=[
            pltpu.VMEM((_NCHH, _K), jnp.int32),
            pltpu.VMEM((_NCHH, _K), jnp.int32),
            pltpu.VMEM((_K, _D), jnp.float32),
            pltpu.VMEM((_K, _D), jnp.float32),
            pltpu.SemaphoreType.DMA,
            pltpu.SemaphoreType.DMA,
            pltpu.VMEM_SHARED((_NP, _D), jnp.float32),
        ],
    )
    def scat_kernel(y_hbm, src_hbm, dst_hbm, zeros_hbm, out_hbm,
                    src_v, dst_v, rows0, rows1, sem0, sem1, z_sh):
        c = lax.axis_index("c")
        s = lax.axis_index("s")
        w = s * _NC + c
        pltpu.sync_copy(zeros_hbm, z_sh.at[pl.ds(s * _RPT, _RPT)])
        plsc.subcore_barrier()

        for h in range(2):
            pltpu.sync_copy(src_hbm.at[w, pl.ds(h * _NCHH, _NCHH)], src_v)
            pltpu.sync_copy(dst_hbm.at[w, pl.ds(h * _NCHH, _NCHH)], dst_v)
            pltpu.async_copy(y_hbm.at[src_v.at[0]], rows0, sem0)

            def body(jj, carry):
                j0 = 2 * jj
                j1 = j0 + 1
                pltpu.make_async_copy(
                    y_hbm.at[src_v.at[j0]], rows0, sem0).wait()
                pltpu.async_copy(y_hbm.at[src_v.at[j1]], rows1, sem1)
                pltpu.sync_copy(rows0, z_sh.at[dst_v.at[j0]], add=True)
                pltpu.make_async_copy(
                    y_hbm.at[src_v.at[j1]], rows1, sem1).wait()

                @pl.when(j0 + 2 < _NCHH)
                def _():
                    pltpu.async_copy(y_hbm.at[src_v.at[j0 + 2]], rows0, sem0)

                pltpu.sync_copy(rows1, z_sh.at[dst_v.at[j1]], add=True)
                return carry

            lax.fori_loop(0, _NCHH // 2, body, 0)
        plsc.subcore_barrier()
        pltpu.sync_copy(z_sh.at[pl.ds(s * _RPT, _RPT)],
                        out_hbm.at[c, pl.ds(s * _RPT, _RPT)])

    return scat_kernel(y, src_r, dst_r, zeros128)


def _dinv_block(deg_ref):
    return lax.rsqrt(deg_ref[0, :, 0:1] + deg_ref[1, :, 0:1] + 1.0)


def _tc_dense1(x, deg, W1, b1, Wg1):
    def body(x_ref, deg_ref, W1_ref, b1_ref, Wg1_ref, y_ref):
        dinv = _dinv_block(deg_ref)
        h = jnp.maximum(
            jnp.dot(x_ref[...], W1_ref[...],
                    preferred_element_type=jnp.float32) + b1_ref[...], 0.0)
        y_ref[...] = dinv * jnp.dot(h, Wg1_ref[...],
                                    preferred_element_type=jnp.float32)

    return pl.pallas_call(
        body,
        grid=(_NBLK,),
        in_specs=[
            pl.BlockSpec((_BLK, _D), lambda i: (i, 0)),
            pl.BlockSpec((_NC, _BLK, _D), lambda i: (0, i, 0)),
            pl.BlockSpec((_D, _D), lambda i: (0, 0)),
            pl.BlockSpec((1, _D), lambda i: (0, 0)),
            pl.BlockSpec((_D, _D), lambda i: (0, 0)),
        ],
        out_specs=pl.BlockSpec((_BLK, _D), lambda i: (i, 0)),
        out_shape=jax.ShapeDtypeStruct((_N, _D), jnp.float32),
    )(x, deg, W1, b1, Wg1)


def _tc_mid(p, yprev, deg, bg, Wgn):
    def body(p_ref, y_ref, deg_ref, bg_ref, Wg_ref, o_ref):
        dinv = _dinv_block(deg_ref)
        z = p_ref[0] + p_ref[1] + y_ref[...]
        h = jnp.maximum(dinv * z + bg_ref[...], 0.0)
        o_ref[...] = dinv * jnp.dot(h, Wg_ref[...],
                                    preferred_element_type=jnp.float32)

    return pl.pallas_call(
        body,
        grid=(_NBLK,),
        in_specs=[
            pl.BlockSpec((_NC, _BLK, _D), lambda i: (0, i, 0)),
            pl.BlockSpec((_BLK, _D), lambda i: (i, 0)),
            pl.BlockSpec((_NC, _BLK, _D), lambda i: (0, i, 0)),
            pl.BlockSpec((1, _D), lambda i: (0, 0)),
            pl.BlockSpec((_D, _D), lambda i: (0, 0)),
        ],
        out_specs=pl.BlockSpec((_BLK, _D), lambda i: (i, 0)),
        out_shape=jax.ShapeDtypeStruct((_N, _D), jnp.float32),
    )(p, yprev, deg, bg, Wgn)


def _tc_pool(p, y3, deg, bg3, batch_r, W2, b2):
    def body(p_ref, y_ref, deg_ref, bg_ref, b_ref, W2_ref, b2_ref,
             o_ref, sums, cnts):
        i = pl.program_id(0)

        @pl.when(i == 0)
        def _():
            sums[...] = jnp.zeros_like(sums)
            cnts[...] = jnp.zeros_like(cnts)

        dinv = _dinv_block(deg_ref)
        h = jnp.maximum(dinv * (p_ref[0] + p_ref[1] + y_ref[...])
                        + bg_ref[...], 0.0)
        gids = lax.broadcasted_iota(jnp.int32, (_G, _BLK), 0)
        onehot = (gids == b_ref[0]).astype(jnp.float32)
        sums[...] += jnp.dot(onehot, h, preferred_element_type=jnp.float32)
        cnts[...] += jnp.broadcast_to(
            jnp.sum(onehot, axis=1, keepdims=True), (_G, _D))

        @pl.when(i == _NBLK - 1)
        def _():
            pooled = sums[...] / jnp.maximum(cnts[...], 1.0)
            o_ref[...] = jnp.dot(pooled, W2_ref[...],
                                 preferred_element_type=jnp.float32) + b2_ref[...]

    return pl.pallas_call(
        body,
        grid=(_NBLK,),
        in_specs=[
            pl.BlockSpec((_NC, _BLK, _D), lambda i: (0, i, 0)),
            pl.BlockSpec((_BLK, _D), lambda i: (i, 0)),
            pl.BlockSpec((_NC, _BLK, _D), lambda i: (0, i, 0)),
            pl.BlockSpec((1, _D), lambda i: (0, 0)),
            pl.BlockSpec((1, 1, _BLK), lambda i: (i, 0, 0)),
            pl.BlockSpec((_D, _D), lambda i: (0, 0)),
            pl.BlockSpec((1, _D), lambda i: (0, 0)),
        ],
        out_specs=pl.BlockSpec((_G, _D), lambda i: (0, 0)),
        out_shape=jax.ShapeDtypeStruct((_G, _D), jnp.float32),
        scratch_shapes=[
            pltpu.VMEM((_G, _D), jnp.float32),
            pltpu.VMEM((_G, _D), jnp.float32),
        ],
    )(p, y3, deg, bg3, batch_r, W2, b2)


def kernel(x, edge_index, batch, W1, b1, Wg1, bg1, Wg2, bg2, Wg3, bg3, W2, b2):
    src = edge_index[0].reshape(_NW, _NCH, _K)
    dst = edge_index[1].reshape(_NW, _NCH, _K)
    zeros128 = jnp.zeros((_RPT, _D), jnp.float32)
    ones128 = jnp.ones((_K, _D), jnp.float32)
    batch_r = batch.reshape(_NBLK, 1, _BLK)
    b1r = b1.reshape(1, _D)
    bg1r = bg1.reshape(1, _D)
    bg2r = bg2.reshape(1, _D)
    bg3r = bg3.reshape(1, _D)
    b2r = b2.reshape(1, _D)

    deg = _sc_degree(dst, ones128, zeros128)
    y1 = _tc_dense1(x, deg, W1, b1r, Wg1)
    p1 = _sc_scatter(y1, src, dst, zeros128)
    y2 = _tc_mid(p1, y1, deg, bg1r, Wg2)
    p2 = _sc_scatter(y2, src, dst, zeros128)
    y3 = _tc_mid(p2, y2, deg, bg2r, Wg3)
    p3 = _sc_scatter(y3, src, dst, zeros128)
    return _tc_pool(p3, y3, deg, bg3r, batch_r, W2, b2r)


# speedup vs baseline: 2.8122x; 1.0156x over previous
import functools

import jax
import jax.numpy as jnp
from jax import lax
from jax.experimental import pallas as pl
from jax.experimental.pallas import tpu as pltpu
from jax.experimental.pallas import tpu_sc as plsc

_N = 10000
_E = 320000
_D = 128
_G = 64
_NC = 2
_NS = 16
_NW = _NC * _NS
_EPW = _E // _NW
_K = 125
_NCH = _EPW // _K
_NCHH = _NCH // 2
_NP = 10112
_RPT = _NP // _NS
_BLK = 1000
_NBLK = _N // _BLK


def _sc_mesh():
    return plsc.VectorSubcoreMesh(
        core_axis_name="c", subcore_axis_name="s",
        num_cores=_NC, num_subcores=_NS)


def _sc_degree(dst_r, ones_rows, zeros128):
    @functools.partial(
        pl.kernel,
        out_type=jax.ShapeDtypeStruct((_NC, _NP, _D), jnp.float32),
        mesh=_sc_mesh(),
        scratch_types=[
            pltpu.VMEM((_NCH, _K), jnp.int32),
            pltpu.VMEM((_K, _D), jnp.float32),
            pltpu.VMEM_SHARED((_NP, _D), jnp.float32),
        ],
    )
    def deg_kernel(dst_hbm, ones_hbm, zeros_hbm, out_hbm, dst_v, ones_v, deg_sh):
        c = lax.axis_index("c")
        s = lax.axis_index("s")
        w = s * _NC + c
        pltpu.sync_copy(dst_hbm.at[w], dst_v)
        pltpu.sync_copy(ones_hbm, ones_v)
        pltpu.sync_copy(zeros_hbm, deg_sh.at[pl.ds(s * _RPT, _RPT)])
        plsc.subcore_barrier()

        def body(j, carry):
            pltpu.sync_copy(ones_v, deg_sh.at[dst_v.at[j]], add=True)
            return carry

        lax.fori_loop(0, _NCH, body, 0)
        plsc.subcore_barrier()
        pltpu.sync_copy(deg_sh.at[pl.ds(s * _RPT, _RPT)],
                        out_hbm.at[c, pl.ds(s * _RPT, _RPT)])

    return deg_kernel(dst_r, ones_rows, zeros128)


def _sc_scatter(y, src_r, dst_r, zeros128):
    @functools.partial(
        pl.kernel,
        out_type=jax.ShapeDtypeStruct((_NC, _NP, _D), jnp.float32),
        mesh=_sc_mesh(),
        scratch_types=[
            pltpu.VMEM((_NCHH, _K), jnp.int32),
            pltpu.VMEM((_NCHH, _K), jnp.int32),
            pltpu.VMEM((_K, _D), jnp.float32),
            pltpu.VMEM((_K, _D), jnp.float32),
            pltpu.SemaphoreType.DMA,
            pltpu.SemaphoreType.DMA,
            pltpu.VMEM_SHARED((_NP, _D), jnp.float32),
        ],
    )
    def scat_kernel(y_hbm, src_hbm, dst_hbm, zeros_hbm, out_hbm,
                    src_v, dst_v, rows0, rows1, sem0, sem1, z_sh):
        c = lax.axis_index("c")
        s = lax.axis_index("s")
        w = s * _NC + c

        @pl.when(c == 0)
        def _():
            @pl.when(s < _NS - 1)
            def _():
                pltpu.sync_copy(y_hbm.at[pl.ds(s * _RPT, _RPT)],
                                z_sh.at[pl.ds(s * _RPT, _RPT)])

            @pl.when(s == _NS - 1)
            def _():
                pltpu.sync_copy(y_hbm.at[pl.ds((_NS - 1) * _RPT, _N - (_NS - 1) * _RPT)],
                                z_sh.at[pl.ds((_NS - 1) * _RPT, _N - (_NS - 1) * _RPT)])

        @pl.when(c == 1)
        def _():
            pltpu.sync_copy(zeros_hbm, z_sh.at[pl.ds(s * _RPT, _RPT)])
        plsc.subcore_barrier()

        for h in range(2):
            pltpu.sync_copy(src_hbm.at[w, pl.ds(h * _NCHH, _NCHH)], src_v)
            pltpu.sync_copy(dst_hbm.at[w, pl.ds(h * _NCHH, _NCHH)], dst_v)
            pltpu.async_copy(y_hbm.at[src_v.at[0]], rows0, sem0)

            def body(jj, carry):
                j0 = 2 * jj
                j1 = j0 + 1
                pltpu.make_async_copy(
                    y_hbm.at[src_v.at[j0]], rows0, sem0).wait()
                pltpu.async_copy(y_hbm.at[src_v.at[j1]], rows1, sem1)
                pltpu.sync_copy(rows0, z_sh.at[dst_v.at[j0]], add=True)
                pltpu.make_async_copy(
                    y_hbm.at[src_v.at[j1]], rows1, sem1).wait()

                @pl.when(j0 + 2 < _NCHH)
                def _():
                    pltpu.async_copy(y_hbm.at[src_v.at[j0 + 2]], rows0, sem0)

                pltpu.sync_copy(rows1, z_sh.at[dst_v.at[j1]], add=True)
                return carry

            lax.fori_loop(0, _NCHH // 2, body, 0)
        plsc.subcore_barrier()
        pltpu.sync_copy(z_sh.at[pl.ds(s * _RPT, _RPT)],
                        out_hbm.at[c, pl.ds(s * _RPT, _RPT)])

    return scat_kernel(y, src_r, dst_r, zeros128)


def _dinv_block(deg_ref):
    return lax.rsqrt(deg_ref[0, :, 0:1] + deg_ref[1, :, 0:1] + 1.0)


def _tc_h(x, W1, b1):
    def body(x_ref, W1_ref, b1_ref, h_ref):
        h_ref[...] = jnp.maximum(
            jnp.dot(x_ref[...], W1_ref[...],
                    preferred_element_type=jnp.float32) + b1_ref[...], 0.0)

    return pl.pallas_call(
        body,
        grid=(_NBLK,),
        in_specs=[
            pl.BlockSpec((_BLK, _D), lambda i: (i, 0)),
            pl.BlockSpec((_D, _D), lambda i: (0, 0)),
            pl.BlockSpec((1, _D), lambda i: (0, 0)),
        ],
        out_specs=pl.BlockSpec((_BLK, _D), lambda i: (i, 0)),
        out_shape=jax.ShapeDtypeStruct((_N, _D), jnp.float32),
    )(x, W1, b1)


def _tc_scale(h, deg, Wg1):
    def body(h_ref, deg_ref, Wg1_ref, y_ref, dv_ref):
        dinv = _dinv_block(deg_ref)
        y_ref[...] = dinv * jnp.dot(h_ref[...], Wg1_ref[...],
                                    preferred_element_type=jnp.float32)
        dv_ref[...] = jnp.broadcast_to(dinv, (_BLK, _D))

    return pl.pallas_call(
        body,
        grid=(_NBLK,),
        in_specs=[
            pl.BlockSpec((_BLK, _D), lambda i: (i, 0)),
            pl.BlockSpec((_NC, _BLK, _D), lambda i: (0, i, 0)),
            pl.BlockSpec((_D, _D), lambda i: (0, 0)),
        ],
        out_specs=[
            pl.BlockSpec((_BLK, _D), lambda i: (i, 0)),
            pl.BlockSpec((_BLK, _D), lambda i: (i, 0)),
        ],
        out_shape=[
            jax.ShapeDtypeStruct((_N, _D), jnp.float32),
            jax.ShapeDtypeStruct((_N, _D), jnp.float32),
        ],
    )(h, deg, Wg1)


def _tc_mid(p, dinv, bg, Wgn):
    def body(p_ref, dv_ref, bg_ref, Wg_ref, o_ref):
        dinv = dv_ref[...]
        z = p_ref[0] + p_ref[1]
        h = jnp.maximum(dinv * z + bg_ref[...], 0.0)
        o_ref[...] = dinv * jnp.dot(h, Wg_ref[...],
                                    preferred_element_type=jnp.float32)

    return pl.pallas_call(
        body,
        grid=(_NBLK,),
        in_specs=[
            pl.BlockSpec((_NC, _BLK, _D), lambda i: (0, i, 0)),
            pl.BlockSpec((_BLK, _D), lambda i: (i, 0)),
            pl.BlockSpec((1, _D), lambda i: (0, 0)),
            pl.BlockSpec((_D, _D), lambda i: (0, 0)),
        ],
        out_specs=pl.BlockSpec((_BLK, _D), lambda i: (i, 0)),
        out_shape=jax.ShapeDtypeStruct((_N, _D), jnp.float32),
    )(p, dinv, bg, Wgn)


def _tc_pool(p, dinv_a, bg3, batch_r, W2, b2):
    def body(p_ref, dv_ref, bg_ref, b_ref, W2_ref, b2_ref,
             o_ref, sums, cnts):
        i = pl.program_id(0)

        @pl.when(i == 0)
        def _():
            sums[...] = jnp.zeros_like(sums)
            cnts[...] = jnp.zeros_like(cnts)

        h = jnp.maximum(dv_ref[...] * (p_ref[0] + p_ref[1])
                        + bg_ref[...], 0.0)
        gids = lax.broadcasted_iota(jnp.int32, (_G, _BLK), 0)
        onehot = (gids == b_ref[0]).astype(jnp.float32)
        sums[...] += jnp.dot(onehot, h, preferred_element_type=jnp.float32)
        cnts[...] += jnp.broadcast_to(
            jnp.sum(onehot, axis=1, keepdims=True), (_G, _D))

        @pl.when(i == _NBLK - 1)
        def _():
            pooled = sums[...] / jnp.maximum(cnts[...], 1.0)
            o_ref[...] = jnp.dot(pooled, W2_ref[...],
                                 preferred_element_type=jnp.float32) + b2_ref[...]

    return pl.pallas_call(
        body,
        grid=(_NBLK,),
        in_specs=[
            pl.BlockSpec((_NC, _BLK, _D), lambda i: (0, i, 0)),
            pl.BlockSpec((_BLK, _D), lambda i: (i, 0)),
            pl.BlockSpec((1, _D), lambda i: (0, 0)),
            pl.BlockSpec((1, 1, _BLK), lambda i: (i, 0, 0)),
            pl.BlockSpec((_D, _D), lambda i: (0, 0)),
            pl.BlockSpec((1, _D), lambda i: (0, 0)),
        ],
        out_specs=pl.BlockSpec((_G, _D), lambda i: (0, 0)),
        out_shape=jax.ShapeDtypeStruct((_G, _D), jnp.float32),
        scratch_shapes=[
            pltpu.VMEM((_G, _D), jnp.float32),
            pltpu.VMEM((_G, _D), jnp.float32),
        ],
    )(p, dinv_a, bg3, batch_r, W2, b2)


def kernel(x, edge_index, batch, W1, b1, Wg1, bg1, Wg2, bg2, Wg3, bg3, W2, b2):
    src = edge_index[0].reshape(_NW, _NCH, _K)
    dst = edge_index[1].reshape(_NW, _NCH, _K)
    zeros128 = jnp.zeros((_RPT, _D), jnp.float32)
    ones128 = jnp.ones((_K, _D), jnp.float32)
    batch_r = batch.reshape(_NBLK, 1, _BLK)
    b1r = b1.reshape(1, _D)
    bg1r = bg1.reshape(1, _D)
    bg2r = bg2.reshape(1, _D)
    bg3r = bg3.reshape(1, _D)
    b2r = b2.reshape(1, _D)

    deg = _sc_degree(dst, ones128, zeros128)
    h = _tc_h(x, W1, b1r)
    y1, dinv = _tc_scale(h, deg, Wg1)
    p1 = _sc_scatter(y1, src, dst, zeros128)
    y2 = _tc_mid(p1, dinv, bg1r, Wg2)
    p2 = _sc_scatter(y2, src, dst, zeros128)
    y3 = _tc_mid(p2, dinv, bg2r, Wg3)
    p3 = _sc_scatter(y3, src, dst, zeros128)
    return _tc_pool(p3, dinv, bg3r, batch_r, W2, b2r)


# speedup vs baseline: 3.1079x; 1.1051x over previous
import functools

import jax
import jax.numpy as jnp
from jax import lax
from jax.experimental import pallas as pl
from jax.experimental.pallas import tpu as pltpu
from jax.experimental.pallas import tpu_sc as plsc

_N = 10000
_E = 320000
_D = 128
_G = 64
_NC = 2
_NS = 16
_NW = _NC * _NS
_EPW = _E // _NW
_K = 125
_NCH = _EPW // _K
_NCHH = _NCH // 2
_NP = 10112
_RPT = _NP // _NS
_HR = 80
_BLK = 1000
_NBLK = _N // _BLK


def _sc_mesh():
    return plsc.VectorSubcoreMesh(
        core_axis_name="c", subcore_axis_name="s",
        num_cores=_NC, num_subcores=_NS)


def _sc_degree(dst_flat, iota_hr, zeros_hr):
    @functools.partial(
        pl.kernel,
        out_type=jax.ShapeDtypeStruct((_NC, _HR, _D), jnp.float32),
        mesh=_sc_mesh(),
        compiler_params=pltpu.CompilerParams(needs_layout_passes=False),
        scratch_types=[
            pltpu.VMEM((_EPW,), jnp.int32),
            pltpu.VMEM((_HR * _D,), jnp.float32),
            pltpu.VMEM((_HR, _D), jnp.float32),
            pltpu.VMEM((_HR,), jnp.int32),
            pltpu.VMEM_SHARED((_HR, _D), jnp.float32),
        ],
    )
    def deg_kernel(dst_hbm, iota_hbm, zeros_hbm, out_hbm,
                   dst_v, hist1, hist2, ident_v, z_sh):
        c = lax.axis_index("c")
        s = lax.axis_index("s")
        w = s * _NC + c
        pltpu.sync_copy(dst_hbm.at[w], dst_v)
        pltpu.sync_copy(iota_hbm, ident_v)

        @pl.when(s == 0)
        def _():
            pltpu.sync_copy(zeros_hbm, z_sh)

        z16 = jnp.zeros((16,), jnp.float32)

        def zbody(i, carry):
            hist1[pl.ds(i * 16, 16)] = z16
            return carry

        lax.fori_loop(0, _HR * _D // 16, zbody, 0)

        ones16 = jnp.ones((16,), jnp.float32)

        def body(i, carry):
            idx = dst_v[pl.ds(i * 16, 16)]
            plsc.addupdate_scatter(hist1, [idx], ones16)
            return carry

        lax.fori_loop(0, _EPW // 16, body, 0)

        def cbody(i, carry):
            for cc in range(_D // 16):
                hist2[i, pl.ds(cc * 16, 16)] = hist1[pl.ds(i * _D + cc * 16, 16)]
            return carry

        lax.fori_loop(0, _HR, cbody, 0)
        plsc.subcore_barrier()
        pltpu.sync_copy(hist2, z_sh.at[ident_v], add=True)
        plsc.subcore_barrier()

        @pl.when(s == 0)
        def _():
            pltpu.sync_copy(z_sh, out_hbm.at[c])

    return deg_kernel(dst_flat, iota_hr, zeros_hr)


def _sc_scatter(y, src_r, dst_r, zeros128):
    @functools.partial(
        pl.kernel,
        out_type=jax.ShapeDtypeStruct((_NC, _NP, _D), jnp.float32),
        mesh=_sc_mesh(),
        scratch_types=[
            pltpu.VMEM((_NCHH, _K), jnp.int32),
            pltpu.VMEM((_NCHH, _K), jnp.int32),
            pltpu.VMEM((_K, _D), jnp.float32),
            pltpu.VMEM((_K, _D), jnp.float32),
            pltpu.SemaphoreType.DMA,
            pltpu.SemaphoreType.DMA,
            pltpu.VMEM_SHARED((_NP, _D), jnp.float32),
        ],
    )
    def scat_kernel(y_hbm, src_hbm, dst_hbm, zeros_hbm, out_hbm,
                    src_v, dst_v, rows0, rows1, sem0, sem1, z_sh):
        c = lax.axis_index("c")
        s = lax.axis_index("s")
        w = s * _NC + c

        @pl.when(c == 0)
        def _():
            @pl.when(s < _NS - 1)
            def _():
                pltpu.sync_copy(y_hbm.at[pl.ds(s * _RPT, _RPT)],
                                z_sh.at[pl.ds(s * _RPT, _RPT)])

            @pl.when(s == _NS - 1)
            def _():
                pltpu.sync_copy(y_hbm.at[pl.ds((_NS - 1) * _RPT, _N - (_NS - 1) * _RPT)],
                                z_sh.at[pl.ds((_NS - 1) * _RPT, _N - (_NS - 1) * _RPT)])

        @pl.when(c == 1)
        def _():
            pltpu.sync_copy(zeros_hbm, z_sh.at[pl.ds(s * _RPT, _RPT)])
        plsc.subcore_barrier()

        for h in range(2):
            pltpu.sync_copy(src_hbm.at[w, pl.ds(h * _NCHH, _NCHH)], src_v)
            pltpu.sync_copy(dst_hbm.at[w, pl.ds(h * _NCHH, _NCHH)], dst_v)
            pltpu.async_copy(y_hbm.at[src_v.at[0]], rows0, sem0)

            def body(jj, carry):
                j0 = 2 * jj
                j1 = j0 + 1
                pltpu.make_async_copy(
                    y_hbm.at[src_v.at[j0]], rows0, sem0).wait()
                pltpu.async_copy(y_hbm.at[src_v.at[j1]], rows1, sem1)
                pltpu.sync_copy(rows0, z_sh.at[dst_v.at[j0]], add=True)
                pltpu.make_async_copy(
                    y_hbm.at[src_v.at[j1]], rows1, sem1).wait()

                @pl.when(j0 + 2 < _NCHH)
                def _():
                    pltpu.async_copy(y_hbm.at[src_v.at[j0 + 2]], rows0, sem0)

                pltpu.sync_copy(rows1, z_sh.at[dst_v.at[j1]], add=True)
                return carry

            lax.fori_loop(0, _NCHH // 2, body, 0)
        plsc.subcore_barrier()
        pltpu.sync_copy(z_sh.at[pl.ds(s * _RPT, _RPT)],
                        out_hbm.at[c, pl.ds(s * _RPT, _RPT)])

    return scat_kernel(y, src_r, dst_r, zeros128)


def _dinv_block(dg0_ref, dg1_ref):
    return lax.rsqrt(dg0_ref[...] + dg1_ref[...] + 1.0)


def _tc_dense1(x, dg0, dg1, W1, b1, Wg1):
    def body(x_ref, dg0_ref, dg1_ref, W1_ref, b1_ref, Wg1_ref, y_ref):
        dinv = _dinv_block(dg0_ref, dg1_ref)
        h = jnp.maximum(
            jnp.dot(x_ref[...], W1_ref[...],
                    preferred_element_type=jnp.float32) + b1_ref[...], 0.0)
        y_ref[...] = dinv * jnp.dot(h, Wg1_ref[...],
                                    preferred_element_type=jnp.float32)

    return pl.pallas_call(
        body,
        grid=(_NBLK,),
        in_specs=[
            pl.BlockSpec((_BLK, _D), lambda i: (i, 0)),
            pl.BlockSpec((_BLK, 1), lambda i: (i, 0)),
            pl.BlockSpec((_BLK, 1), lambda i: (i, 0)),
            pl.BlockSpec((_D, _D), lambda i: (0, 0)),
            pl.BlockSpec((1, _D), lambda i: (0, 0)),
            pl.BlockSpec((_D, _D), lambda i: (0, 0)),
        ],
        out_specs=pl.BlockSpec((_BLK, _D), lambda i: (i, 0)),
        out_shape=jax.ShapeDtypeStruct((_N, _D), jnp.float32),
    )(x, dg0, dg1, W1, b1, Wg1)


def _tc_mid(p, dg0, dg1, bg, Wgn):
    def body(p_ref, dg0_ref, dg1_ref, bg_ref, Wg_ref, o_ref):
        dinv = _dinv_block(dg0_ref, dg1_ref)
        z = p_ref[0] + p_ref[1]
        h = jnp.maximum(dinv * z + bg_ref[...], 0.0)
        o_ref[...] = dinv * jnp.dot(h, Wg_ref[...],
                                    preferred_element_type=jnp.float32)

    return pl.pallas_call(
        body,
        grid=(_NBLK,),
        in_specs=[
            pl.BlockSpec((_NC, _BLK, _D), lambda i: (0, i, 0)),
            pl.BlockSpec((_BLK, 1), lambda i: (i, 0)),
            pl.BlockSpec((_BLK, 1), lambda i: (i, 0)),
            pl.BlockSpec((1, _D), lambda i: (0, 0)),
            pl.BlockSpec((_D, _D), lambda i: (0, 0)),
        ],
        out_specs=pl.BlockSpec((_BLK, _D), lambda i: (i, 0)),
        out_shape=jax.ShapeDtypeStruct((_N, _D), jnp.float32),
    )(p, dg0, dg1, bg, Wgn)


def _tc_pool(p, dg0, dg1, bg3, batch_r, W2, b2):
    def body(p_ref, dg0_ref, dg1_ref, bg_ref, b_ref, W2_ref, b2_ref,
             o_ref, sums, cnts):
        i = pl.program_id(0)

        @pl.when(i == 0)
        def _():
            sums[...] = jnp.zeros_like(sums)
            cnts[...] = jnp.zeros_like(cnts)

        dinv = _dinv_block(dg0_ref, dg1_ref)
        h = jnp.maximum(dinv * (p_ref[0] + p_ref[1])
                        + bg_ref[...], 0.0)
        gids = lax.broadcasted_iota(jnp.int32, (_G, _BLK), 0)
        onehot = (gids == b_ref[0]).astype(jnp.float32)
        sums[...] += jnp.dot(onehot, h, preferred_element_type=jnp.float32)
        cnts[...] += jnp.broadcast_to(
            jnp.sum(onehot, axis=1, keepdims=True), (_G, _D))

        @pl.when(i == _NBLK - 1)
        def _():
            pooled = sums[...] / jnp.maximum(cnts[...], 1.0)
            o_ref[...] = jnp.dot(pooled, W2_ref[...],
                                 preferred_element_type=jnp.float32) + b2_ref[...]

    return pl.pallas_call(
        body,
        grid=(_NBLK,),
        in_specs=[
            pl.BlockSpec((_NC, _BLK, _D), lambda i: (0, i, 0)),
            pl.BlockSpec((_BLK, 1), lambda i: (i, 0)),
            pl.BlockSpec((_BLK, 1), lambda i: (i, 0)),
            pl.BlockSpec((1, _D), lambda i: (0, 0)),
            pl.BlockSpec((1, 1, _BLK), lambda i: (i, 0, 0)),
            pl.BlockSpec((_D, _D), lambda i: (0, 0)),
            pl.BlockSpec((1, _D), lambda i: (0, 0)),
        ],
        out_specs=pl.BlockSpec((_G, _D), lambda i: (0, 0)),
        out_shape=jax.ShapeDtypeStruct((_G, _D), jnp.float32),
        scratch_shapes=[
            pltpu.VMEM((_G, _D), jnp.float32),
            pltpu.VMEM((_G, _D), jnp.float32),
        ],
    )(p, dg0, dg1, bg3, batch_r, W2, b2)


def kernel(x, edge_index, batch, W1, b1, Wg1, bg1, Wg2, bg2, Wg3, bg3, W2, b2):
    src = edge_index[0].reshape(_NW, _NCH, _K)
    dst = edge_index[1].reshape(_NW, _NCH, _K)
    dstf = edge_index[1].reshape(_NW, _EPW)
    zeros128 = jnp.zeros((_RPT, _D), jnp.float32)
    zeros_hr = jnp.zeros((_HR, _D), jnp.float32)
    iota_hr = jnp.arange(_HR, dtype=jnp.int32)
    batch_r = batch.reshape(_NBLK, 1, _BLK)
    b1r = b1.reshape(1, _D)
    bg1r = bg1.reshape(1, _D)
    bg2r = bg2.reshape(1, _D)
    bg3r = bg3.reshape(1, _D)
    b2r = b2.reshape(1, _D)

    deg = _sc_degree(dstf, iota_hr, zeros_hr)
    dg0 = deg[0].reshape(_HR * _D, 1)[:_N]
    dg1 = deg[1].reshape(_HR * _D, 1)[:_N]
    y1 = _tc_dense1(x, dg0, dg1, W1, b1r, Wg1)
    p1 = _sc_scatter(y1, src, dst, zeros128)
    y2 = _tc_mid(p1, dg0, dg1, bg1r, Wg2)
    p2 = _sc_scatter(y2, src, dst, zeros128)
    y3 = _tc_mid(p2, dg0, dg1, bg2r, Wg3)
    p3 = _sc_scatter(y3, src, dst, zeros128)
    return _tc_pool(p3, dg0, dg1, bg3r, batch_r, W2, b2r)
